# baseline (speedup 1.0000x reference)
import functools

import jax
import jax.numpy as jnp
from jax import lax
from jax.experimental import pallas as pl
from jax.experimental.pallas import tpu as pltpu
from jax.experimental.pallas import tpu_sc as plsc

B, L, D = 1024, 200, 64
NUM_LABELS = 100000

_NC, _NS = 2, 16
_NW = _NC * _NS
_RPW = B // _NW


_WV = 8
_NWAVE = L // _WV


def _bag_body(x_hbm, xw_hbm, table_hbm, out_hbm, idx_all, w_all, grp_a, grp_b,
              out_v, sem_a, sem_b):
    wid = lax.axis_index("s") * _NC + lax.axis_index("c")
    base = wid * _RPW * L
    pltpu.sync_copy(x_hbm.at[pl.ds(base, _RPW * L)],
                    idx_all.at[pl.ds(0, _RPW * L)])
    pltpu.sync_copy(xw_hbm.at[pl.ds(base, _RPW * L)],
                    w_all.at[pl.ds(0, _RPW * L)])

    cols = [lax.iota(jnp.int32, 16) + 16 * k for k in range(4)]

    def fire(r, c, grp, sem):
        idx16 = idx_all[pl.ds(r * L + _WV * c, 16)]
        for u in range(_WV):
            g = pl.multiple_of((idx16[u] // 8) * 8, 8)
            pltpu.make_async_copy(
                table_hbm.at[pl.ds(g, 8)], grp.at[u], sem).start()

    def compute(r, c, grp, sem, accs):
        idx16w = idx_all[pl.ds(r * L + _WV * c, 16)]
        for u in range(_WV):
            g = pl.multiple_of((idx16w[u] // 8) * 8, 8)
            pltpu.make_async_copy(
                table_hbm.at[pl.ds(g, 8)], grp.at[u], sem).wait()
        idx16 = idx_all[pl.ds(r * L + _WV * c, 16)]
        w16 = w_all[pl.ds(r * L + _WV * c, 16)]
        g16 = lax.rem(idx16, jnp.full((16,), 8, jnp.int32))
        for u in range(_WV):
            u16 = jnp.full((16,), u, jnp.int32)
            r16 = jnp.full((16,), g16[u], jnp.int32)
            w = w16[u]
            accs = tuple(
                a + w * plsc.load_gather(grp, [u16, r16, cols[k]])
                for k, a in enumerate(accs))
        return accs

    def row_step(r):
        z = jnp.zeros((16,), jnp.float32)
        fire(r, 0, grp_a, sem_a)

        def unit(k, accs):
            accs = compute(r, 2 * k, grp_a, sem_a,
                           _fire_then(r, 2 * k + 1, grp_b, sem_b, accs))
            accs = compute(r, 2 * k + 1, grp_b, sem_b,
                           _fire_next(r, 2 * k + 2, grp_a, sem_a, accs))
            return accs

        def _fire_then(r, c, grp, sem, accs):
            fire(r, c, grp, sem)
            return accs

        def _fire_next(r, c, grp, sem, accs):
            @pl.when(c < _NWAVE)
            def _():
                fire(r, c, grp, sem)
            return accs

        accs = lax.fori_loop(0, _NWAVE // 2, unit, (z, z, z, z))
        a0, a1, a2, a3 = compute(r, _NWAVE - 1, grp_a, sem_a, accs)
        out_v[pl.ds(r * D, 16)] = a0
        out_v[pl.ds(r * D + 16, 16)] = a1
        out_v[pl.ds(r * D + 32, 16)] = a2
        out_v[pl.ds(r * D + 48, 16)] = a3

    def row(r, carry):
        row_step(r)
        return carry

    lax.fori_loop(0, _RPW, row, 0)
    pltpu.sync_copy(out_v, out_hbm.at[pl.ds(wid * _RPW * D, _RPW * D)])


_bag = functools.partial(
    pl.kernel,
    mesh=plsc.VectorSubcoreMesh(core_axis_name="c", subcore_axis_name="s"),
    compiler_params=pltpu.CompilerParams(needs_layout_passes=False),
    out_type=jax.ShapeDtypeStruct((B * D,), jnp.float32),
    scratch_types=[
        pltpu.VMEM((_RPW * L + 16,), jnp.int32),
        pltpu.VMEM((_RPW * L + 16,), jnp.float32),
        pltpu.VMEM((_WV, 8, D), jnp.float32),
        pltpu.VMEM((_WV, 8, D), jnp.float32),
        pltpu.VMEM((_RPW * D,), jnp.float32),
        pltpu.SemaphoreType.DMA,
        pltpu.SemaphoreType.DMA,
    ],
)(_bag_body)


_BL = 2048


def _cls_body(emb_ref, wt_ref, bt_ref, wc_ref, bc_ref, out_ref, h_ref):
    @pl.when(pl.program_id(0) == 0)
    def _():
        h = jnp.dot(emb_ref[...], wt_ref[...], preferred_element_type=jnp.float32)
        h_ref[...] = jnp.maximum(h + bt_ref[...], 0.0)

    out_ref[...] = lax.dot_general(
        h_ref[...], wc_ref[...],
        dimension_numbers=(((1,), (1,)), ((), ())),
        preferred_element_type=jnp.float32,
    ) + bc_ref[...]


_classify = pl.pallas_call(
    _cls_body,
    grid=(pl.cdiv(NUM_LABELS, _BL),),
    in_specs=[
        pl.BlockSpec((B, D), lambda j: (0, 0)),
        pl.BlockSpec((D, D), lambda j: (0, 0)),
        pl.BlockSpec((1, D), lambda j: (0, 0)),
        pl.BlockSpec((_BL, D), lambda j: (j, 0)),
        pl.BlockSpec((1, _BL), lambda j: (0, j)),
    ],
    out_specs=pl.BlockSpec((B, _BL), lambda j: (0, j)),
    out_shape=jax.ShapeDtypeStruct((B, NUM_LABELS), jnp.float32),
    scratch_shapes=[pltpu.VMEM((B, D), jnp.float32)],
)


def kernel(X, X_w, emb_table, W_t, b_t, W_c, b_c):
    embed = _bag(X.reshape(B * L), X_w.reshape(B * L), emb_table)
    return _classify(embed.reshape(B, D), W_t, b_t.reshape(1, D),
                     W_c, b_c.reshape(1, NUM_LABELS))


# speedup vs baseline: 1.1934x; 1.1934x over previous
import functools

import jax
import jax.numpy as jnp
from jax import lax
from jax.experimental import pallas as pl
from jax.experimental.pallas import tpu as pltpu
from jax.experimental.pallas import tpu_sc as plsc

B, L, D = 1024, 200, 64
NUM_LABELS = 100000

_NC, _NS = 2, 16
_NW = _NC * _NS
_RPW = B // _NW


_WV = 8
_NWAVE = L // _WV


def _bag_body(x_hbm, xw_hbm, table_hbm, out_hbm, idx_all, w_all, grp_a, grp_b,
              out_v, sem_a, sem_b):
    wid = lax.axis_index("s") * _NC + lax.axis_index("c")
    base = wid * _RPW * L
    pltpu.sync_copy(x_hbm.at[pl.ds(base, _RPW * L)],
                    idx_all.at[pl.ds(0, _RPW * L)])
    pltpu.sync_copy(xw_hbm.at[pl.ds(base, _RPW * L)],
                    w_all.at[pl.ds(0, _RPW * L)])

    cols = [lax.iota(jnp.int32, 16) + 16 * k for k in range(4)]

    def fire(r, c, grp, sem):
        idx16 = idx_all[pl.ds(r * L + _WV * c, 16)]
        for u in range(_WV):
            pltpu.make_async_copy(
                table_hbm.at[idx16[u]], grp.at[u], sem).start()

    def compute(r, c, grp, sem, accs):
        idx16 = idx_all[pl.ds(r * L + _WV * c, 16)]
        for u in range(_WV):
            pltpu.make_async_copy(
                table_hbm.at[idx16[u]], grp.at[u], sem).wait()
        w16 = w_all[pl.ds(r * L + _WV * c, 16)]
        for u in range(_WV):
            u16 = jnp.full((16,), u, jnp.int32)
            w = w16[u]
            accs = tuple(
                a + w * plsc.load_gather(grp, [u16, cols[k]])
                for k, a in enumerate(accs))
        return accs

    def row_step(r):
        z = jnp.zeros((16,), jnp.float32)
        fire(r, 0, grp_a, sem_a)

        def unit(k, accs):
            accs = compute(r, 2 * k, grp_a, sem_a,
                           _fire_then(r, 2 * k + 1, grp_b, sem_b, accs))
            accs = compute(r, 2 * k + 1, grp_b, sem_b,
                           _fire_next(r, 2 * k + 2, grp_a, sem_a, accs))
            return accs

        def _fire_then(r, c, grp, sem, accs):
            fire(r, c, grp, sem)
            return accs

        def _fire_next(r, c, grp, sem, accs):
            @pl.when(c < _NWAVE)
            def _():
                fire(r, c, grp, sem)
            return accs

        accs = lax.fori_loop(0, _NWAVE // 2, unit, (z, z, z, z))
        a0, a1, a2, a3 = compute(r, _NWAVE - 1, grp_a, sem_a, accs)
        out_v[pl.ds(r * D, 16)] = a0
        out_v[pl.ds(r * D + 16, 16)] = a1
        out_v[pl.ds(r * D + 32, 16)] = a2
        out_v[pl.ds(r * D + 48, 16)] = a3

    def row(r, carry):
        row_step(r)
        return carry

    lax.fori_loop(0, _RPW, row, 0)
    pltpu.sync_copy(out_v, out_hbm.at[pl.ds(wid * _RPW * D, _RPW * D)])


_bag = functools.partial(
    pl.kernel,
    mesh=plsc.VectorSubcoreMesh(core_axis_name="c", subcore_axis_name="s"),
    compiler_params=pltpu.CompilerParams(needs_layout_passes=False),
    out_type=jax.ShapeDtypeStruct((B * D,), jnp.float32),
    scratch_types=[
        pltpu.VMEM((_RPW * L + 16,), jnp.int32),
        pltpu.VMEM((_RPW * L + 16,), jnp.float32),
        pltpu.VMEM((_WV, D), jnp.float32),
        pltpu.VMEM((_WV, D), jnp.float32),
        pltpu.VMEM((_RPW * D,), jnp.float32),
        pltpu.SemaphoreType.DMA,
        pltpu.SemaphoreType.DMA,
    ],
)(_bag_body)


_BL = 2048


def _cls_body(emb_ref, wt_ref, bt_ref, wc_ref, bc_ref, out_ref, h_ref):
    @pl.when(pl.program_id(0) == 0)
    def _():
        h = jnp.dot(emb_ref[...], wt_ref[...], preferred_element_type=jnp.float32)
        h_ref[...] = jnp.maximum(h + bt_ref[...], 0.0)

    out_ref[...] = lax.dot_general(
        h_ref[...], wc_ref[...],
        dimension_numbers=(((1,), (1,)), ((), ())),
        preferred_element_type=jnp.float32,
    ) + bc_ref[...]


_classify = pl.pallas_call(
    _cls_body,
    grid=(pl.cdiv(NUM_LABELS, _BL),),
    in_specs=[
        pl.BlockSpec((B, D), lambda j: (0, 0)),
        pl.BlockSpec((D, D), lambda j: (0, 0)),
        pl.BlockSpec((1, D), lambda j: (0, 0)),
        pl.BlockSpec((_BL, D), lambda j: (j, 0)),
        pl.BlockSpec((1, _BL), lambda j: (0, j)),
    ],
    out_specs=pl.BlockSpec((B, _BL), lambda j: (0, j)),
    out_shape=jax.ShapeDtypeStruct((B, NUM_LABELS), jnp.float32),
    scratch_shapes=[pltpu.VMEM((B, D), jnp.float32)],
)


def kernel(X, X_w, emb_table, W_t, b_t, W_c, b_c):
    embed = _bag(X.reshape(B * L), X_w.reshape(B * L), emb_table)
    return _classify(embed.reshape(B, D), W_t, b_t.reshape(1, D),
                     W_c, b_c.reshape(1, NUM_LABELS))


# speedup vs baseline: 1.4499x; 1.2149x over previous
import functools

import jax
import jax.numpy as jnp
from jax import lax
from jax.experimental import pallas as pl
from jax.experimental.pallas import tpu as pltpu
from jax.experimental.pallas import tpu_sc as plsc

B, L, D = 1024, 200, 64
NUM_LABELS = 100000

_NC, _NS = 2, 16
_NW = _NC * _NS
_RPW = B // _NW


_WV = 8
_NWAVE = L // _WV


def _bag_body(x_hbm, xw_hbm, table_hbm, out_hbm, idx_all, w_all, grp_a, grp_b,
              out_v, sem_a, sem_b):
    wid = lax.axis_index("s") * _NC + lax.axis_index("c")
    base = wid * _RPW * L
    pltpu.sync_copy(x_hbm.at[pl.ds(base, _RPW * L)],
                    idx_all.at[pl.ds(0, _RPW * L)])
    pltpu.sync_copy(xw_hbm.at[pl.ds(base, _RPW * L)],
                    w_all.at[pl.ds(0, _RPW * L)])

    cols = [lax.iota(jnp.int32, 16) + 16 * k for k in range(4)]

    def row_dma(r, buf, sem, is_start):
        def op(l, x):
            cp = pltpu.make_async_copy(table_hbm.at[x], buf.at[l], sem)
            cp.start() if is_start else cp.wait()

        def fchunk(c, carry):
            idx16 = idx_all[pl.ds(r * L + 16 * c, 16)]
            for u in range(16):
                op(16 * c + u, idx16[u])
            return carry

        lax.fori_loop(0, 12, fchunk, 0)
        idx16 = idx_all[pl.ds(r * L + L - 16, 16)]
        for u in range(8, 16):
            op(L - 16 + u, idx16[u])

    def compute_row(r, buf):
        z = jnp.zeros((16,), jnp.float32)

        def fma(l, w, accs):
            l16 = jnp.full((16,), l, jnp.int32)
            return tuple(
                a + w * plsc.load_gather(buf, [l16, cols[k]])
                for k, a in enumerate(accs))

        def chunk(c, accs):
            w16 = w_all[pl.ds(r * L + 16 * c, 16)]
            for u in range(16):
                accs = fma(16 * c + u, w16[u], accs)
            return accs

        accs = lax.fori_loop(0, 12, chunk, (z, z, z, z))
        w16 = w_all[pl.ds(r * L + L - 16, 16)]
        for u in range(8, 16):
            accs = fma(L - 16 + u, w16[u], accs)
        a0, a1, a2, a3 = accs
        out_v[pl.ds(r * D, 16)] = a0
        out_v[pl.ds(r * D + 16, 16)] = a1
        out_v[pl.ds(r * D + 32, 16)] = a2
        out_v[pl.ds(r * D + 48, 16)] = a3

    def row_step(r, buf, sem):
        row_dma(r, buf, sem, False)
        compute_row(r, buf)

        @pl.when(r + 2 < _RPW)
        def _():
            row_dma(r + 2, buf, sem, True)

    row_dma(0, grp_a, sem_a, True)
    row_dma(1, grp_b, sem_b, True)

    def pair(p, carry):
        row_step(2 * p, grp_a, sem_a)
        row_step(2 * p + 1, grp_b, sem_b)
        return carry

    lax.fori_loop(0, _RPW // 2, pair, 0)
    pltpu.sync_copy(out_v, out_hbm.at[pl.ds(wid * _RPW * D, _RPW * D)])


_bag = functools.partial(
    pl.kernel,
    mesh=plsc.VectorSubcoreMesh(core_axis_name="c", subcore_axis_name="s"),
    compiler_params=pltpu.CompilerParams(needs_layout_passes=False),
    out_type=jax.ShapeDtypeStruct((B * D,), jnp.float32),
    scratch_types=[
        pltpu.VMEM((_RPW * L + 16,), jnp.int32),
        pltpu.VMEM((_RPW * L + 16,), jnp.float32),
        pltpu.VMEM((L, D), jnp.float32),
        pltpu.VMEM((L, D), jnp.float32),
        pltpu.VMEM((_RPW * D,), jnp.float32),
        pltpu.SemaphoreType.DMA,
        pltpu.SemaphoreType.DMA,
    ],
)(_bag_body)


_BL = 2048


def _cls_body(emb_ref, wt_ref, bt_ref, wc_ref, bc_ref, out_ref, h_ref):
    @pl.when(pl.program_id(0) == 0)
    def _():
        h = jnp.dot(emb_ref[...], wt_ref[...], preferred_element_type=jnp.float32)
        h_ref[...] = jnp.maximum(h + bt_ref[...], 0.0)

    out_ref[...] = lax.dot_general(
        h_ref[...], wc_ref[...],
        dimension_numbers=(((1,), (1,)), ((), ())),
        preferred_element_type=jnp.float32,
    ) + bc_ref[...]


_classify = pl.pallas_call(
    _cls_body,
    grid=(pl.cdiv(NUM_LABELS, _BL),),
    in_specs=[
        pl.BlockSpec((B, D), lambda j: (0, 0)),
        pl.BlockSpec((D, D), lambda j: (0, 0)),
        pl.BlockSpec((1, D), lambda j: (0, 0)),
        pl.BlockSpec((_BL, D), lambda j: (j, 0)),
        pl.BlockSpec((1, _BL), lambda j: (0, j)),
    ],
    out_specs=pl.BlockSpec((B, _BL), lambda j: (0, j)),
    out_shape=jax.ShapeDtypeStruct((B, NUM_LABELS), jnp.float32),
    scratch_shapes=[pltpu.VMEM((B, D), jnp.float32)],
)


def kernel(X, X_w, emb_table, W_t, b_t, W_c, b_c):
    embed = _bag(X.reshape(B * L), X_w.reshape(B * L), emb_table)
    return _classify(embed.reshape(B, D), W_t, b_t.reshape(1, D),
                     W_c, b_c.reshape(1, NUM_LABELS))
